# baseline (speedup 1.0000x reference)
import functools

import jax
import jax.numpy as jnp
from jax import lax
from jax.experimental import pallas as pl
from jax.experimental.pallas import tpu as pltpu
from jax.experimental.pallas import tpu_sc as plsc

_N = 100000
_E = 6400000
_T = 16

_NC, _NS = 2, 16
_NW = _NC * _NS
_NPAD = 100096
_ROWS_P2 = _NPAD // 128
_PER_TILE = _NPAD // _NS
_WSTRIDE = 200064
_CHUNK = 2560
_NCHUNK = 80
_NPAIR = _NCHUNK // 2


@functools.cache
def _build_phase1():
    mesh = plsc.VectorSubcoreMesh(
        core_axis_name="c", subcore_axis_name="s", num_cores=_NC, num_subcores=_NS
    )
    return functools.partial(
        pl.kernel,
        out_type=jax.ShapeDtypeStruct((_NC * _NPAD,), jnp.int32),
        mesh=mesh,
        scratch_types=[
            pltpu.VMEM((_PER_TILE,), jnp.int32),
            pltpu.VMEM((_CHUNK,), jnp.int32),
            pltpu.VMEM((_CHUNK,), jnp.int32),
            pltpu.VMEM((_CHUNK,), jnp.int32),
            pltpu.VMEM_SHARED((_NPAD,), jnp.int32),
            pltpu.SemaphoreType.DMA,
            pltpu.SemaphoreType.DMA,
        ],
    )(_phase1_body)


def _phase1_body(edge_hbm, out_hbm, stage_v, idx_a, idx_b, ones_v, flags_sh, sem_a, sem_b):
    c = lax.axis_index("c")
    s = lax.axis_index("s")
    wid = s * _NC + c

    def _zero(i, carry):
        stage_v[pl.ds(i * 16, 16)] = jnp.zeros((16,), jnp.int32)
        return carry

    lax.fori_loop(jnp.int32(0), jnp.int32(_PER_TILE // 16), _zero, 0)
    pltpu.sync_copy(stage_v, flags_sh.at[pl.ds(s * _PER_TILE, _PER_TILE)])

    def _one(i, carry):
        ones_v[pl.ds(i * 16, 16)] = jnp.ones((16,), jnp.int32)
        return carry

    lax.fori_loop(jnp.int32(0), jnp.int32(_CHUNK // 16), _one, 0)
    plsc.subcore_barrier()

    wstart = wid * _WSTRIDE

    def _base(k):
        b = jnp.minimum(wstart + k * _CHUNK, _E - _CHUNK)
        return pl.multiple_of(b, 128)

    def _start(buf, sem, k):
        pltpu.async_copy(edge_hbm.at[jnp.int32(0), pl.ds(_base(k), _CHUNK)], buf, sem)

    def _wait(buf, sem):
        pltpu.make_async_copy(
            edge_hbm.at[jnp.int32(0), pl.ds(jnp.int32(0), _CHUNK)], buf, sem
        ).wait()

    _start(idx_a, sem_a, jnp.int32(0))

    def _pair(p, carry):
        _start(idx_b, sem_b, 2 * p + 1)
        _wait(idx_a, sem_a)
        pltpu.sync_copy(ones_v, flags_sh.at[idx_a])
        _start(idx_a, sem_a, 2 * p + 2)
        _wait(idx_b, sem_b)
        pltpu.sync_copy(ones_v, flags_sh.at[idx_b])
        return carry

    lax.fori_loop(jnp.int32(0), jnp.int32(_NPAIR), _pair, 0)
    _wait(idx_a, sem_a)
    plsc.subcore_barrier()

    pltpu.sync_copy(flags_sh.at[pl.ds(s * _PER_TILE, _PER_TILE)], stage_v)
    pltpu.sync_copy(stage_v, out_hbm.at[pl.ds(c * _NPAD + s * _PER_TILE, _PER_TILE)])


def _phase2_body(f_ref, sp_ref, cnt_ref, std_ref, bias_ref, o_ref):
    f = f_ref[...]
    sp = sp_ref[...]
    center = (cnt_ref[0] > 0) | (cnt_ref[1] > 0)
    sg = jnp.zeros_like(f)
    bg = jnp.zeros_like(f)
    for t in range(_T):
        m = sp == t
        sg = sg + jnp.where(m, std_ref[t], 0.0)
        bg = bg + jnp.where(m, bias_ref[t], 0.0)
    o_ref[...] = jnp.where(center, f * sg + bg, f)


def kernel(node_features, edge_index, node_type, per_type_std, per_type_bias):
    edge_src = lax.bitcast_convert_type(edge_index.astype(jnp.uint32), jnp.int32)
    flags = _build_phase1()(edge_src)

    f_pad = jnp.pad(node_features[:, 0], (0, _NPAD - _N)).reshape(_ROWS_P2, 128)
    sp_pad = jnp.pad(node_type[:, 0].astype(jnp.int32), (0, _NPAD - _N)).reshape(
        _ROWS_P2, 128
    )
    cnt3 = flags.reshape(_NC, _ROWS_P2, 128)

    out2 = pl.pallas_call(
        _phase2_body,
        out_shape=jax.ShapeDtypeStruct((_ROWS_P2, 128), jnp.float32),
        in_specs=[
            pl.BlockSpec(memory_space=pltpu.VMEM),
            pl.BlockSpec(memory_space=pltpu.VMEM),
            pl.BlockSpec(memory_space=pltpu.VMEM),
            pl.BlockSpec(memory_space=pltpu.SMEM),
            pl.BlockSpec(memory_space=pltpu.SMEM),
        ],
    )(f_pad, sp_pad, cnt3, per_type_std[:, 0], per_type_bias[:, 0])

    return out2.reshape(_NPAD)[:_N].reshape(_N, 1)


# speedup vs baseline: 1.0170x; 1.0170x over previous
import functools

import jax
import jax.numpy as jnp
from jax import lax
from jax.experimental import pallas as pl
from jax.experimental.pallas import tpu as pltpu
from jax.experimental.pallas import tpu_sc as plsc

_N = 100000
_E = 6400000
_T = 16

_NC, _NS = 2, 16
_NW = _NC * _NS
_NPAD = 100096
_ROWS_P2 = _NPAD // 128
_PER_TILE = _NPAD // _NS
_WSTRIDE = 200064
_CHUNK = 5120
_NCHUNK = 40
_NPAIR = _NCHUNK // 2


@functools.cache
def _build_phase1():
    mesh = plsc.VectorSubcoreMesh(
        core_axis_name="c", subcore_axis_name="s", num_cores=_NC, num_subcores=_NS
    )
    return functools.partial(
        pl.kernel,
        out_type=jax.ShapeDtypeStruct((_NC * _NPAD,), jnp.int32),
        mesh=mesh,
        scratch_types=[
            pltpu.VMEM((_PER_TILE,), jnp.int32),
            pltpu.VMEM((_CHUNK,), jnp.int32),
            pltpu.VMEM((_CHUNK,), jnp.int32),
            pltpu.VMEM((_CHUNK,), jnp.int32),
            pltpu.VMEM_SHARED((_NPAD,), jnp.int32),
            pltpu.SemaphoreType.DMA,
            pltpu.SemaphoreType.DMA,
        ],
    )(_phase1_body)


def _phase1_body(edge_hbm, out_hbm, stage_v, idx_a, idx_b, ones_v, flags_sh, sem_a, sem_b):
    c = lax.axis_index("c")
    s = lax.axis_index("s")
    wid = s * _NC + c

    def _zero(i, carry):
        stage_v[pl.ds(i * 16, 16)] = jnp.zeros((16,), jnp.int32)
        return carry

    lax.fori_loop(jnp.int32(0), jnp.int32(_PER_TILE // 16), _zero, 0)
    pltpu.sync_copy(stage_v, flags_sh.at[pl.ds(s * _PER_TILE, _PER_TILE)])

    def _one(i, carry):
        ones_v[pl.ds(i * 16, 16)] = jnp.ones((16,), jnp.int32)
        return carry

    lax.fori_loop(jnp.int32(0), jnp.int32(_CHUNK // 16), _one, 0)
    plsc.subcore_barrier()

    wstart = wid * _WSTRIDE

    def _base(k):
        b = jnp.minimum(wstart + k * _CHUNK, _E - _CHUNK)
        return pl.multiple_of(b, 128)

    def _start(buf, sem, k):
        pltpu.async_copy(edge_hbm.at[pl.ds(_base(k), _CHUNK)], buf, sem)

    def _wait(buf, sem):
        pltpu.make_async_copy(
            edge_hbm.at[pl.ds(jnp.int32(0), _CHUNK)], buf, sem
        ).wait()

    _start(idx_a, sem_a, jnp.int32(0))

    def _pair(p, carry):
        _start(idx_b, sem_b, 2 * p + 1)
        _wait(idx_a, sem_a)
        pltpu.sync_copy(ones_v, flags_sh.at[idx_a])
        _start(idx_a, sem_a, 2 * p + 2)
        _wait(idx_b, sem_b)
        pltpu.sync_copy(ones_v, flags_sh.at[idx_b])
        return carry

    lax.fori_loop(jnp.int32(0), jnp.int32(_NPAIR), _pair, 0)
    _wait(idx_a, sem_a)
    plsc.subcore_barrier()

    pltpu.sync_copy(flags_sh.at[pl.ds(s * _PER_TILE, _PER_TILE)], stage_v)
    pltpu.sync_copy(stage_v, out_hbm.at[pl.ds(c * _NPAD + s * _PER_TILE, _PER_TILE)])


def _phase2_body(f_ref, sp_ref, cnt_ref, std_ref, bias_ref, o_ref):
    f = f_ref[...]
    sp = sp_ref[...]
    center = (cnt_ref[0] > 0) | (cnt_ref[1] > 0)
    sg = jnp.zeros_like(f)
    bg = jnp.zeros_like(f)
    for t in range(_T):
        m = sp == t
        sg = sg + jnp.where(m, std_ref[t], 0.0)
        bg = bg + jnp.where(m, bias_ref[t], 0.0)
    o_ref[...] = jnp.where(center, f * sg + bg, f)


def kernel(node_features, edge_index, node_type, per_type_std, per_type_bias):
    edge_src = edge_index[0].astype(jnp.int32)
    flags = _build_phase1()(edge_src)

    f_pad = jnp.pad(node_features[:, 0], (0, _NPAD - _N)).reshape(_ROWS_P2, 128)
    sp_pad = jnp.pad(node_type[:, 0].astype(jnp.int32), (0, _NPAD - _N)).reshape(
        _ROWS_P2, 128
    )
    cnt3 = flags.reshape(_NC, _ROWS_P2, 128)

    out2 = pl.pallas_call(
        _phase2_body,
        out_shape=jax.ShapeDtypeStruct((_ROWS_P2, 128), jnp.float32),
        in_specs=[
            pl.BlockSpec(memory_space=pltpu.VMEM),
            pl.BlockSpec(memory_space=pltpu.VMEM),
            pl.BlockSpec(memory_space=pltpu.VMEM),
            pl.BlockSpec(memory_space=pltpu.SMEM),
            pl.BlockSpec(memory_space=pltpu.SMEM),
        ],
    )(f_pad, sp_pad, cnt3, per_type_std[:, 0], per_type_bias[:, 0])

    return out2.reshape(_NPAD)[:_N].reshape(_N, 1)
